# baseline (speedup 1.0000x reference)
import functools

import jax
import jax.numpy as jnp
from jax import lax
from jax.experimental import pallas as pl
from jax.experimental.pallas import tpu as pltpu
from jax.experimental.pallas import tpu_sc as plsc

_D = 32
_IN_W = 17
_B = 16384
_S = 50
_SP = 56
_LANES = 128

_NW = 32
_SUPER = _B // 128 // _NW
_NBC = 8
_NSUB = 128 // _NBC


def _body(xp_hbm, tab_hbm, ids_hbm, out_hbm, xv, idsv, idxb, outv, sem):
    wid = lax.axis_index("s") * 2 + lax.axis_index("c")

    def super_body(g, carry):
        sb = (wid + g * _NW) * 128
        pltpu.sync_copy(ids_hbm.at[:, pl.ds(sb, 128)], idsv)

        def sub_body(k, c2):
            b0 = sb + k * _NBC
            bo = k * _NBC
            pltpu.sync_copy(xp_hbm.at[pl.ds(b0, _NBC)], xv)

            def idx_build(sl, c3):
                idxb[pl.ds(sl * 8, 16)] = idsv[sl, pl.ds(bo, 16)].astype(
                    jnp.int32)
                return c3

            lax.fori_loop(0, _SP, idx_build, 0)

            def fire(sl, c3):
                pltpu.async_copy(tab_hbm.at[idxb.at[pl.ds(sl * 8, 8)]],
                                 outv.at[:, sl, :], sem)
                return c3

            def drain(sl, c3):
                pltpu.make_async_copy(
                    tab_hbm.at[idxb.at[pl.ds(0, 8)]],
                    outv.at[:, sl, :], sem).wait()
                return c3

            lax.fori_loop(0, _SP, fire, 0)
            lax.fori_loop(0, _SP, drain, 0)

            for bl in range(_NBC):
                def feats(sl, c3, bl=bl):
                    outv[bl, sl, pl.ds(_D, 16)] = xv[bl, sl, pl.ds(1, 16)]
                    return c3

                lax.fori_loop(0, _SP, feats, 0)

            pltpu.sync_copy(outv, out_hbm.at[pl.ds(b0, _NBC)])
            return c2

        lax.fori_loop(0, _NSUB, sub_body, 0)
        return carry

    lax.fori_loop(0, _SUPER, super_body, 0)


@functools.partial(
    pl.kernel,
    mesh=plsc.VectorSubcoreMesh(core_axis_name="c", subcore_axis_name="s"),
    out_type=jax.ShapeDtypeStruct((_B, _SP, _LANES), jnp.float32),
    scratch_types=[
        pltpu.VMEM((_NBC, _SP, _IN_W), jnp.float32),
        pltpu.VMEM((_SP, 128), jnp.float32),
        pltpu.VMEM((_SP * 8 + 8,), jnp.int32),
        pltpu.VMEM((_NBC, _SP, _LANES), jnp.float32),
        pltpu.SemaphoreType.DMA,
    ],
)
def _sc_embed(xp_hbm, tab_hbm, ids_hbm, out_hbm, xv, idsv, idxb, outv, sem):
    _body(xp_hbm, tab_hbm, ids_hbm, out_hbm, xv, idsv, idxb, outv, sem)


def kernel(x, embed_weight):
    xp = jnp.pad(x, ((0, 0), (0, _SP - _S), (0, 0)))
    tabp = jnp.pad(embed_weight, ((0, 0), (0, _LANES - _D)))
    idsp = jnp.pad(x[:, :, 0], ((0, 0), (0, _SP - _S))).T
    outp = _sc_embed(xp, tabp, idsp)
    return outp[:, :_S, :_D + 16]


# speedup vs baseline: 5.5998x; 5.5998x over previous
import functools

import jax
import jax.numpy as jnp
from jax import lax
from jax.experimental import pallas as pl
from jax.experimental.pallas import tpu as pltpu
from jax.experimental.pallas import tpu_sc as plsc

_D = 32
_B = 16384
_S = 50
_SP = 56
_LANES = 128

_NW = 32
_BW = 256
_NCH = _B // _BW
_MAIN_UNITS = 6 * _NCH
_TAIL_UNITS = _NCH


def _body(tab_hbm, ids_hbm, out_hbm, idsv, idxb, ev, sem0, sem1):
    wid = lax.axis_index("s") * 2 + lax.axis_index("c")
    sems = (sem0, sem1)

    def run_unit(u, nsl):
        blk = u // _NCH
        s0 = blk * 8
        b0 = (u % _NCH) * _BW
        pltpu.sync_copy(ids_hbm.at[pl.ds(s0, 8), pl.ds(b0, _BW)], idsv)

        for sl in range(nsl + 1):
            p = sl % 2
            if sl < nsl:
                def idx_build(i, c3, sl=sl, p=p):
                    idxb[p, pl.ds(i * 16, 16)] = idsv[
                        sl, pl.ds(i * 16, 16)].astype(jnp.int32)
                    return c3

                lax.fori_loop(0, _BW // 16, idx_build, 0)
                for t in range(_BW // 128):
                    pltpu.async_copy(
                        tab_hbm.at[idxb.at[p, pl.ds(t * 128, 128)]],
                        ev.at[p, pl.ds(t * 128, 128), :], sems[p])
            if sl >= 1:
                q = (sl - 1) % 2
                for t in range(_BW // 128):
                    pltpu.make_async_copy(
                        tab_hbm.at[idxb.at[q, pl.ds(0, 128)]],
                        ev.at[q, pl.ds(t * 128, 128), :], sems[q]).wait()
                row0 = (s0 + (sl - 1)) * _B + b0
                pltpu.sync_copy(ev.at[q], out_hbm.at[pl.ds(row0, _BW)])

    def main_unit(k, carry):
        run_unit(wid + k * _NW, 8)
        return carry

    def tail_unit(k, carry):
        run_unit(6 * _NCH + wid + k * _NW, 2)
        return carry

    lax.fori_loop(0, _MAIN_UNITS // _NW, main_unit, 0)
    lax.fori_loop(0, _TAIL_UNITS // _NW, tail_unit, 0)


@functools.partial(
    pl.kernel,
    mesh=plsc.VectorSubcoreMesh(core_axis_name="c", subcore_axis_name="s"),
    out_type=jax.ShapeDtypeStruct((_S * _B, _LANES), jnp.float32),
    scratch_types=[
        pltpu.VMEM((8, _BW), jnp.float32),
        pltpu.VMEM((2, _BW), jnp.int32),
        pltpu.VMEM((2, _BW, _LANES), jnp.float32),
        pltpu.SemaphoreType.DMA,
        pltpu.SemaphoreType.DMA,
    ],
)
def _sc_embed(tab_hbm, ids_hbm, out_hbm, idsv, idxb, ev, sem0, sem1):
    _body(tab_hbm, ids_hbm, out_hbm, idsv, idxb, ev, sem0, sem1)


def kernel(x, embed_weight):
    tabp = jnp.pad(embed_weight, ((0, 0), (0, _LANES - _D)))
    idsp = jnp.pad(x[:, :, 0], ((0, 0), (0, _SP - _S))).T
    embf = _sc_embed(tabp, idsp)
    emb = embf.reshape(_S, _B, _LANES)[:, :, :_D].transpose(1, 0, 2)
    return jnp.concatenate([emb, x[:, :, 1:]], axis=-1)


# speedup vs baseline: 5.6006x; 1.0001x over previous
import functools

import jax
import jax.numpy as jnp
from jax import lax
from jax.experimental import pallas as pl
from jax.experimental.pallas import tpu as pltpu
from jax.experimental.pallas import tpu_sc as plsc

_D = 32
_B = 16384
_S = 50
_SP = 56
_LANES = 128

_NW = 32
_BW = 256
_NCH = _B // _BW
_MAIN_UNITS = 6 * _NCH
_TAIL_UNITS = _NCH


def _body(tab_hbm, ids_hbm, out_hbm, idsv, idxb, ev, sem0, sem1, osem):
    wid = lax.axis_index("s") * 2 + lax.axis_index("c")
    sems = (sem0, sem1)

    def run_unit(u, nsl):
        blk = u // _NCH
        s0 = blk * 8
        b0 = (u % _NCH) * _BW
        pltpu.sync_copy(ids_hbm.at[pl.ds(s0, 8), pl.ds(b0, _BW)], idsv)

        def wait_out():
            pltpu.make_async_copy(
                ev.at[0], out_hbm.at[pl.ds(0, _BW)], osem).wait()

        for sl in range(nsl + 1):
            p = sl % 2
            if sl < nsl:
                if sl >= 2:
                    wait_out()
                def idx_build(i, c3, sl=sl, p=p):
                    idxb[p, pl.ds(i * 16, 16)] = idsv[
                        sl, pl.ds(i * 16, 16)].astype(jnp.int32)
                    return c3

                lax.fori_loop(0, _BW // 16, idx_build, 0)
                for t in range(_BW // 128):
                    pltpu.async_copy(
                        tab_hbm.at[idxb.at[p, pl.ds(t * 128, 128)]],
                        ev.at[p, pl.ds(t * 128, 128), :], sems[p])
            if sl >= 1:
                q = (sl - 1) % 2
                for t in range(_BW // 128):
                    pltpu.make_async_copy(
                        tab_hbm.at[idxb.at[q, pl.ds(0, 128)]],
                        ev.at[q, pl.ds(t * 128, 128), :], sems[q]).wait()
                row0 = (s0 + (sl - 1)) * _B + b0
                pltpu.async_copy(ev.at[q], out_hbm.at[pl.ds(row0, _BW)],
                                 osem)
        wait_out()
        if nsl >= 2:
            wait_out()

    def main_unit(k, carry):
        run_unit(wid + k * _NW, 8)
        return carry

    def tail_unit(k, carry):
        run_unit(6 * _NCH + wid + k * _NW, 2)
        return carry

    lax.fori_loop(0, _MAIN_UNITS // _NW, main_unit, 0)
    lax.fori_loop(0, _TAIL_UNITS // _NW, tail_unit, 0)


@functools.partial(
    pl.kernel,
    mesh=plsc.VectorSubcoreMesh(core_axis_name="c", subcore_axis_name="s"),
    out_type=jax.ShapeDtypeStruct((_S * _B, _LANES), jnp.float32),
    scratch_types=[
        pltpu.VMEM((8, _BW), jnp.float32),
        pltpu.VMEM((2, _BW), jnp.int32),
        pltpu.VMEM((2, _BW, _LANES), jnp.float32),
        pltpu.SemaphoreType.DMA,
        pltpu.SemaphoreType.DMA,
        pltpu.SemaphoreType.DMA,
    ],
)
def _sc_embed(tab_hbm, ids_hbm, out_hbm, idsv, idxb, ev, sem0, sem1, osem):
    _body(tab_hbm, ids_hbm, out_hbm, idsv, idxb, ev, sem0, sem1, osem)


def kernel(x, embed_weight):
    tabp = jnp.pad(embed_weight, ((0, 0), (0, _LANES - _D)))
    idsp = jnp.pad(x[:, :, 0], ((0, 0), (0, _SP - _S))).T
    embf = _sc_embed(tabp, idsp)
    emb = embf.reshape(_S, _B, _LANES)[:, :, :_D].transpose(1, 0, 2)
    return jnp.concatenate([emb, x[:, :, 1:]], axis=-1)
